# baseline (speedup 1.0000x reference)
import functools

import numpy as np
import jax
import jax.numpy as jnp
from jax.experimental import pallas as pl
from jax.experimental.pallas import tpu as pltpu

_H = 128
_B = 1024


def _plan(n_nodes):
    max_level = int(np.floor(np.log2(n_nodes)))
    levels = []
    for lvl in range(max_level + 1):
        s = 2 ** lvl - 1
        e = min(2 ** (lvl + 1) - 1, n_nodes)
        levels.append((s, e - s))
    return max_level, levels


def _mega_body(tbl_ref, xa_ref, xb_ref, wwt_ref, wb_ref, uf0_ref, uf1_ref,
               uh0_ref, uh1_ref, out_ref, l10_ref, scre_ref, scro_ref,
               c_ref, *, zbase, nsteps):
    i = pl.program_id(0)
    wbase = tbl_ref[i, 1]
    rbase = tbl_ref[i, 2]
    row_limit = tbl_ref[i, 3]

    @pl.when(i == 0)
    def _():
        scre_ref[pl.ds(zbase, _B), :] = jnp.zeros((_B, _H), jnp.float32)
        scro_ref[pl.ds(zbase, _B), :] = jnp.zeros((_B, _H), jnp.float32)

    xa = xa_ref[...]
    xb = xb_ref[...]
    xl = jnp.concatenate([xa[7:, :], xb[:_B - 1, :]], axis=0)
    wx = jnp.dot(xl, wwt_ref[...],
                 preferred_element_type=jnp.float32) + wb_ref[0:1, :]
    whx = wx[:, :_H]
    wfx = wx[:, _H:]

    h0 = scre_ref[pl.ds(rbase, _B), :]
    h1 = scro_ref[pl.ds(rbase, _B), :]

    fpre = (jnp.dot(h0, uf0_ref[...], preferred_element_type=jnp.float32) +
            jnp.dot(h1, uf1_ref[...], preferred_element_type=jnp.float32))
    f = 0.5 * jnp.tanh(0.5 * (fpre + wfx)) + 0.5
    hcand = jnp.tanh(whx +
                     jnp.dot(f * h0, uh0_ref[...],
                             preferred_element_type=jnp.float32) +
                     jnp.dot(f * h1, uh1_ref[...],
                             preferred_element_type=jnp.float32))
    hn = hcand + f * (h0 + h1 - hcand)
    rows = jax.lax.broadcasted_iota(jnp.int32, (_B, 1), 0)
    hn = jnp.where(rows < row_limit, hn, 0.0)

    hsplit = hn.reshape(_B // 2, 2, _H)
    scre_ref[pl.ds(wbase, _B // 2), :] = hsplit[:, 0, :]
    scro_ref[pl.ds(wbase, _B // 2), :] = hsplit[:, 1, :]

    prev0 = c_ref[0:1, :]
    out_ref[...] = jnp.concatenate([hn[1:, :], prev0], axis=0)
    c_ref[0:1, :] = hn[0:1, :]

    @pl.when(i == nsteps - 1)
    def _():
        l10_ref[...] = hn


def _top_body(x_ref, hbuf_ref, slab_ref, wwt_ref, wb_ref, uf0_ref, uf1_ref,
              uh0_ref, uh1_ref, out_ref, *, levels):
    del hbuf_ref
    xb = x_ref[...]
    wwt = wwt_ref[...]
    wb = wb_ref[0:1, :]
    uf0 = uf0_ref[...]
    uf1 = uf1_ref[...]
    uh0 = uh0_ref[...]
    uh1 = uh1_ref[...]
    hp = slab_ref[...]
    out_ref[pl.ds(_B - 1, 1), :] = hp[0:1, :]
    for l in range(9, -1, -1):
        s, n = levels[l]
        np8 = max(16, n)
        need = 2 * np8
        if hp.shape[0] < need:
            hp = jnp.concatenate(
                [hp, jnp.zeros((need - hp.shape[0], _H), jnp.float32)], axis=0)
        pairs = hp[:need].reshape(np8, 2, _H)
        h0 = pairs[:, 0, :]
        h1 = pairs[:, 1, :]
        xl = xb[s:s + np8, :]
        wx = jnp.dot(xl, wwt, preferred_element_type=jnp.float32) + wb
        whx = wx[:, :_H]
        wfx = wx[:, _H:]
        fpre = (jnp.dot(h0, uf0, preferred_element_type=jnp.float32) +
                jnp.dot(h1, uf1, preferred_element_type=jnp.float32))
        f = 0.5 * jnp.tanh(0.5 * (fpre + wfx)) + 0.5
        hcand = jnp.tanh(whx +
                         jnp.dot(f * h0, uh0,
                                 preferred_element_type=jnp.float32) +
                         jnp.dot(f * h1, uh1,
                                 preferred_element_type=jnp.float32))
        hn = hcand + f * (h0 + h1 - hcand)
        rows = jax.lax.broadcasted_iota(jnp.int32, (np8, 1), 0)
        hn = jnp.where(rows < n, hn, 0.0)
        out_ref[pl.ds(s, n), :] = hn[:n, :]
        hp = hn


def kernel(x, W_w, W_b, U_f, U_h):
    n_nodes = x.shape[0]
    max_level, levels = _plan(n_nodes)
    assert max_level >= 10

    wwt = W_w.T
    wb8 = jnp.tile(W_b[None, :], (8, 1))
    uf0 = U_f[:, :_H].T
    uf1 = U_f[:, _H:].T
    uh0 = U_h[:, :_H].T
    uh1 = U_h[:, _H:].T

    nbs = {lvl: -(-levels[lvl][1] // _B) for lvl in range(max_level, 9, -1)}

    capE = max(nbs[lvl] * _B // 2 for lvl in range(max_level, 9, -1))
    zbase = 2 * capE

    tbl = []
    for lvl in range(max_level, 9, -1):
        s, n = levels[lvl]
        pcur = (max_level - lvl) % 2
        child_half = nbs[lvl + 1] * _B // 2 if lvl < max_level else 0
        for j in range(nbs[lvl] - 1, -1, -1):
            xk = (s + 1) // _B + j
            hasc = lvl < max_level and (j + 1) * _B <= child_half
            row_limit = min(_B, n - j * _B)
            tbl.append([xk, pcur * capE + j * (_B // 2),
                        ((1 - pcur) * capE + j * _B) if hasc else zbase,
                        row_limit])
    tbl = np.asarray(tbl, dtype=np.int32)
    nsteps = tbl.shape[0]

    grid_spec = pltpu.PrefetchScalarGridSpec(
        num_scalar_prefetch=1,
        grid=(nsteps,),
        in_specs=[
            pl.BlockSpec((8, _H), lambda i, t: (128 * t[i, 0] - 1, 0)),
            pl.BlockSpec((_B, _H),
                         lambda i, t, m=(n_nodes - 1) // _B:
                         (jnp.minimum(t[i, 0], m), 0)),
            pl.BlockSpec((_H, 2 * _H), lambda i, t: (0, 0)),
            pl.BlockSpec((8, 2 * _H), lambda i, t: (0, 0)),
            pl.BlockSpec((_H, _H), lambda i, t: (0, 0)),
            pl.BlockSpec((_H, _H), lambda i, t: (0, 0)),
            pl.BlockSpec((_H, _H), lambda i, t: (0, 0)),
            pl.BlockSpec((_H, _H), lambda i, t: (0, 0)),
        ],
        out_specs=[
            pl.BlockSpec((_B, _H), lambda i, t: (t[i, 0], 0)),
            pl.BlockSpec((_B, _H), lambda i, t: (0, 0)),
        ],
        scratch_shapes=[
            pltpu.VMEM((2 * capE + _B, _H), jnp.float32),
            pltpu.VMEM((2 * capE + _B, _H), jnp.float32),
            pltpu.VMEM((8, _H), jnp.float32),
        ],
    )

    h_buf, lvl10 = pl.pallas_call(
        functools.partial(_mega_body, zbase=zbase, nsteps=nsteps),
        grid_spec=grid_spec,
        out_shape=[jax.ShapeDtypeStruct((n_nodes, _H), jnp.float32),
                   jax.ShapeDtypeStruct((_B, _H), jnp.float32)],
    )(tbl, x, x, wwt, wb8, uf0, uf1, uh0, uh1)

    out = pl.pallas_call(
        functools.partial(_top_body, levels=tuple(levels)),
        grid=(1,),
        in_specs=[
            pl.BlockSpec((_B, _H), lambda i: (0, 0)),
            pl.BlockSpec((_B, _H), lambda i: (0, 0)),
            pl.BlockSpec((_B, _H), lambda i: (0, 0)),
            pl.BlockSpec((_H, 2 * _H), lambda i: (0, 0)),
            pl.BlockSpec((8, 2 * _H), lambda i: (0, 0)),
            pl.BlockSpec((_H, _H), lambda i: (0, 0)),
            pl.BlockSpec((_H, _H), lambda i: (0, 0)),
            pl.BlockSpec((_H, _H), lambda i: (0, 0)),
            pl.BlockSpec((_H, _H), lambda i: (0, 0)),
        ],
        out_specs=pl.BlockSpec((_B, _H), lambda i: (0, 0)),
        out_shape=jax.ShapeDtypeStruct((n_nodes, _H), jnp.float32),
        input_output_aliases={1: 0},
    )(x, h_buf, lvl10, wwt, wb8, uf0, uf1, uh0, uh1)
    return out
